# baseline (speedup 1.0000x reference)
import functools

import jax
import jax.numpy as jnp
from jax import lax
from jax.experimental import pallas as pl
from jax.experimental.pallas import tpu as pltpu
from jax.experimental.pallas import tpu_sc as plsc

N = 10000
E = 320000
D = 128
R = 400
RP = 512
W = 144
NB = 10
BN = N // NB
NC, NS = 2, 16
NW = NC * NS
BK = 64
NBLK = E // BK
FULL = NBLK // NW
EXTRA = NBLK % NW
PAIRS = FULL // 2
NPAD = 10112
ROWS = NPAD // NS
DW = 16


def _dense_body(ent_ref, ie_ref, tw_ref, tb_ref, tf_ref, fw_ref, fb_ref,
                ir_ref, asrc_ref, adst_ref, wrel_ref,
                x_ref, p_ref, rf_ref):
    f32 = jnp.float32
    tf = tf_ref[...]
    fw = fw_ref[...]
    wf = jnp.sum(fw.reshape(-1, 1, 1) * tf, axis=0)
    w2 = jnp.dot(tw_ref[...], wf[1:], preferred_element_type=f32)
    c0 = wf[0] + jnp.dot(tb_ref[...].reshape(1, -1), wf[1:],
                         preferred_element_type=f32)[0] + fb_ref[...][0]
    fused = jnp.dot(ent_ref[...], w2, preferred_element_type=f32) + c0
    x = ie_ref[...] * fused
    x_ref[...] = x
    u = jnp.concatenate(
        [ir_ref[...] * asrc_ref[...].reshape(1, -1),
         adst_ref[...].reshape(1, -1),
         jnp.zeros((RP - R - 1, D), f32)], axis=0)
    p_ref[...] = jnp.dot(x, u.T, preferred_element_type=f32)

    @pl.when(pl.program_id(0) == 0)
    def _():
        rf_ref[...] = jnp.dot(ir_ref[...], wrel_ref[...],
                              preferred_element_type=f32)


def _dense_stage(ent, init_embed, text_W, text_b, text_factor, fusion_weights,
                 fusion_bias, init_rel, att_src, att_dst, conv_Wrel):
    rep = lambda shape: pl.BlockSpec(shape, lambda i: (0,) * len(shape))
    return pl.pallas_call(
        _dense_body,
        grid=(NB,),
        in_specs=[
            pl.BlockSpec((BN, 768), lambda i: (i, 0)),
            pl.BlockSpec((BN, D), lambda i: (i, 0)),
            rep((768, D)), rep((D,)), rep((16, D + 1, D)), rep((1, 16)),
            rep((1, D)), rep((R, D)), rep((D,)), rep((D,)), rep((D, D)),
        ],
        out_specs=[
            pl.BlockSpec((BN, D), lambda i: (i, 0)),
            pl.BlockSpec((BN, RP), lambda i: (i, 0)),
            pl.BlockSpec((R, D), lambda i: (0, 0)),
        ],
        out_shape=[
            jax.ShapeDtypeStruct((N, D), jnp.float32),
            jax.ShapeDtypeStruct((N, RP), jnp.float32),
            jax.ShapeDtypeStruct((R, D), jnp.float32),
        ],
    )(ent, init_embed, text_W, text_b, text_factor, fusion_weights,
      fusion_bias, init_rel, att_src, att_dst, conv_Wrel)


_MESH = plsc.VectorSubcoreMesh(core_axis_name="c", subcore_axis_name="s")


def _slot_scratch():
    return [
        pltpu.VMEM((3, BK), jnp.int32),
        pltpu.VMEM((BK,), jnp.float32),
        pltpu.VMEM((BK,), jnp.int32),
        pltpu.VMEM((BK,), jnp.int32),
        pltpu.VMEM((BK,), jnp.int32),
        pltpu.VMEM((BK,), jnp.float32),
        pltpu.VMEM((BK,), jnp.float32),
        pltpu.VMEM((BK, D), jnp.float32),
        pltpu.VMEM((BK, D), jnp.float32),
        pltpu.VMEM((BK, DW), jnp.float32),
        pltpu.SemaphoreType.DMA,
        pltpu.SemaphoreType.DMA,
        pltpu.SemaphoreType.DMA,
        pltpu.SemaphoreType.DMA,
        pltpu.SemaphoreType.DMA,
        pltpu.SemaphoreType.DMA,
    ]


@functools.partial(
    pl.kernel, mesh=_MESH,
    compiler_params=pltpu.CompilerParams(use_tc_tiling_on_sc=False),
    out_type=[pltpu.HBM((NC, NPAD, D), jnp.float32),
              pltpu.HBM((NC, NPAD, DW), jnp.float32)],
    scratch_types=[
        pltpu.VMEM_SHARED((R, D), jnp.float32),
        pltpu.VMEM_SHARED((NPAD, D), jnp.float32),
        pltpu.VMEM_SHARED((NPAD, DW), jnp.float32),
    ] + _slot_scratch() + _slot_scratch(),
)
def _edge_pass(ed_hbm, en_hbm, pflat_hbm, x_hbm, rel_hbm, agg_hbm, den_hbm,
               rel_sh, agg_sh, den_sh, *slots):
    f32 = jnp.float32
    cid = lax.axis_index("c")
    sid = lax.axis_index("s")
    wid = sid * NC + cid
    iota = lax.iota(jnp.int32, 16)
    S = [slots[:16], slots[16:]]

    @pl.when(sid == 0)
    def _():
        pltpu.sync_copy(rel_hbm, rel_sh)

    xb0, eb0 = S[0][7], S[0][9]

    def _zxrow(r, _):
        for j in range(D // 16):
            xb0[r, pl.ds(16 * j, 16)] = jnp.zeros((16,), f32)
        eb0[r, pl.ds(0, DW)] = jnp.zeros((16,), f32)
        return _
    lax.fori_loop(0, BK, _zxrow, None)
    base = sid * ROWS
    for i in range(ROWS // BK):
        pltpu.sync_copy(xb0, agg_sh.at[pl.ds(base + i * BK, BK)])
        pltpu.sync_copy(eb0, den_sh.at[pl.ds(base + i * BK, BK)])
    rem = ROWS % BK
    if rem:
        off = base + (ROWS // BK) * BK
        pltpu.sync_copy(xb0.at[pl.ds(0, rem)], agg_sh.at[pl.ds(off, rem)])
        pltpu.sync_copy(eb0.at[pl.ds(0, rem)], den_sh.at[pl.ds(off, rem)])
    plsc.subcore_barrier()

    def _wait_scat(s):
        ed, en, di, i1, i2, g1, g2, xb, rb, eb = S[s][:10]
        sa, sd = S[s][14], S[s][15]
        pltpu.make_async_copy(xb, agg_sh.at[di], sa).wait()
        pltpu.make_async_copy(eb, den_sh.at[di], sd).wait()

    def _prefetch(s, bid):
        ed, en, di, i1, i2, g1, g2, xb, rb, eb = S[s][:10]
        pltpu.sync_copy(ed_hbm.at[:, pl.ds(bid * BK, BK)], ed)
        pltpu.sync_copy(en_hbm.at[pl.ds(bid * BK, BK)], en)
        for g in range(BK // 16):
            sl = pl.ds(16 * g, 16)
            s16 = ed[0, sl]
            d16 = ed[1, sl]
            t16 = ed[2, sl]
            di[sl] = d16
            i1[sl] = s16 * RP + t16
            i2[sl] = d16 * RP + R
        pltpu.async_copy(pflat_hbm.at[i1], g1, S[s][10])
        pltpu.async_copy(pflat_hbm.at[i2], g2, S[s][11])
        pltpu.async_copy(x_hbm.at[ed.at[0]], xb, S[s][12])
        pltpu.async_copy(rel_sh.at[ed.at[2]], rb, S[s][13])

    def _process(s):
        ed, en, di, i1, i2, g1, g2, xb, rb, eb = S[s][:10]
        pltpu.make_async_copy(pflat_hbm.at[i1], g1, S[s][10]).wait()
        pltpu.make_async_copy(pflat_hbm.at[i2], g2, S[s][11]).wait()
        pltpu.make_async_copy(x_hbm.at[ed.at[0]], xb, S[s][12]).wait()
        pltpu.make_async_copy(rel_sh.at[ed.at[2]], rb, S[s][13]).wait()

        @plsc.parallel_loop(0, BK // 16, 1, unroll=2)
        def _group(g):
            k0 = pl.multiple_of(16 * g, 16)
            sl = pl.ds(k0, 16)
            sc = g1[sl] + g2[sl]
            sc = jnp.where(sc >= 0.0, sc, 0.2 * sc)
            e16 = jnp.exp(sc)
            c16 = e16 * en[sl]
            for i in range(16):
                k = k0 + i
                ck = jnp.full((16,), c16[i], f32)
                for j in range(D // 16):
                    slj = pl.ds(16 * j, 16)
                    xb[k, slj] = (ck * xb[k, slj]) * rb[k, slj]
                eb[k, pl.ds(0, DW)] = jnp.where(iota == 0, e16[i], 0.0)
        pltpu.async_copy(xb, agg_sh.at[di], S[s][14], add=True)
        pltpu.async_copy(eb, den_sh.at[di], S[s][15], add=True)

    nextra = (wid < EXTRA).astype(jnp.int32)
    _prefetch(0, wid)

    def _pair(p, _):
        @pl.when(p > 0)
        def _():
            _wait_scat(1)
        _prefetch(1, (2 * p + 1) * NW + wid)
        _process(0)

        @pl.when((2 * p + 2 < FULL) | ((2 * p + 2 == FULL) & (nextra == 1)))
        def _():
            _wait_scat(0)
            _prefetch(0, (2 * p + 2) * NW + wid)
        _process(1)
        return _
    lax.fori_loop(0, PAIRS, _pair, None)

    @pl.when(nextra == 1)
    def _():
        _process(0)
    _wait_scat(0)
    _wait_scat(1)

    plsc.subcore_barrier()
    pltpu.sync_copy(agg_sh.at[pl.ds(base, ROWS)],
                    agg_hbm.at[cid, pl.ds(base, ROWS)])
    pltpu.sync_copy(den_sh.at[pl.ds(base, ROWS)],
                    den_hbm.at[cid, pl.ds(base, ROWS)])


def _out_body(a_ref, d_ref, w_ref, b_ref, o_ref):
    f32 = jnp.float32
    num = a_ref[0] + a_ref[1]
    den = d_ref[0, :, :1] + d_ref[1, :, :1] + 1e-16
    o_ref[...] = jnp.tanh(
        jnp.dot(num / den, w_ref[...], preferred_element_type=f32)
        + b_ref[...])


def _out_stage(agg, den, conv_W, conv_b):
    return pl.pallas_call(
        _out_body,
        grid=(NB,),
        in_specs=[
            pl.BlockSpec((NC, BN, D), lambda i: (0, i, 0)),
            pl.BlockSpec((NC, BN, DW), lambda i: (0, i, 0)),
            pl.BlockSpec((D, D), lambda i: (0, 0)),
            pl.BlockSpec((D,), lambda i: (0,)),
        ],
        out_specs=pl.BlockSpec((BN, D), lambda i: (i, 0)),
        out_shape=jax.ShapeDtypeStruct((N, D), jnp.float32),
    )(agg, den, conv_W, conv_b)


_B = 1024
_BW = _B // NW


@functools.partial(
    pl.kernel, mesh=_MESH,
    compiler_params=pltpu.CompilerParams(use_tc_tiling_on_sc=False),
    out_type=[jax.ShapeDtypeStruct((_B, D), jnp.float32),
              jax.ShapeDtypeStruct((_B, D), jnp.float32)],
    scratch_types=[
        pltpu.VMEM((_BW,), jnp.int32),
        pltpu.VMEM((_BW, D), jnp.float32),
        pltpu.VMEM((_BW,), jnp.int32),
        pltpu.VMEM((_BW, D), jnp.float32),
        pltpu.SemaphoreType.DMA,
        pltpu.SemaphoreType.DMA,
    ],
)
def _gather_pass(xout_hbm, rfull_hbm, subj_hbm, rel_hbm, o1_hbm, o2_hbm,
                 i1_v, r1_v, i2_v, r2_v, sem0, sem1):
    cid = lax.axis_index("c")
    sid = lax.axis_index("s")
    wid = sid * NC + cid
    b0 = wid * _BW
    pltpu.sync_copy(subj_hbm.at[pl.ds(b0, _BW)], i1_v)
    pltpu.sync_copy(rel_hbm.at[pl.ds(b0, _BW)], i2_v)
    d1 = pltpu.async_copy(xout_hbm.at[i1_v], r1_v, sem0)
    d2 = pltpu.async_copy(rfull_hbm.at[i2_v], r2_v, sem1)
    d1.wait()
    d2.wait()
    pltpu.sync_copy(r1_v, o1_hbm.at[pl.ds(b0, _BW)])
    pltpu.sync_copy(r2_v, o2_hbm.at[pl.ds(b0, _BW)])


def kernel(edge_index, edge_type, subj, rel, edge_norm, init_embed,
           ent2textvector, text_W, text_b, text_factor, fusion_weights,
           fusion_bias, init_rel, conv_W, conv_b, conv_Wrel, att_src, att_dst):
    x, p, r_full = _dense_stage(
        ent2textvector, init_embed, text_W, text_b, text_factor,
        fusion_weights, fusion_bias, init_rel, att_src, att_dst, conv_Wrel)
    ed = jnp.concatenate([edge_index, edge_type[None]], axis=0)
    agg, den = _edge_pass(ed, edge_norm, p.reshape(-1), x, init_rel)
    x_out = _out_stage(agg, den, conv_W, conv_b)
    o1, o2 = _gather_pass(x_out, r_full, subj, rel)
    return (o1, o2, x_out)


# speedup vs baseline: 1.0432x; 1.0432x over previous
import functools

import jax
import jax.numpy as jnp
from jax import lax
from jax.experimental import pallas as pl
from jax.experimental.pallas import tpu as pltpu
from jax.experimental.pallas import tpu_sc as plsc

N = 10000
E = 320000
D = 128
R = 400
RP = 512
W = 144
NB = 10
BN = N // NB
NC, NS = 2, 16
NW = NC * NS
BK = 64
NBLK = E // BK
FULL = NBLK // NW
EXTRA = NBLK % NW
PAIRS = FULL // 2
NPAD = 10112
ROWS = NPAD // NS
DW = 16


def _dense_body(ent_ref, ie_ref, tw_ref, tb_ref, tf_ref, fw_ref, fb_ref,
                ir_ref, asrc_ref, adst_ref, wrel_ref,
                x_ref, p_ref, rf_ref):
    f32 = jnp.float32
    tf = tf_ref[...]
    fw = fw_ref[...]
    wf = jnp.sum(fw.reshape(-1, 1, 1) * tf, axis=0)
    w2 = jnp.dot(tw_ref[...], wf[1:], preferred_element_type=f32)
    c0 = wf[0] + jnp.dot(tb_ref[...].reshape(1, -1), wf[1:],
                         preferred_element_type=f32)[0] + fb_ref[...][0]
    fused = jnp.dot(ent_ref[...], w2, preferred_element_type=f32) + c0
    x = ie_ref[...] * fused
    x_ref[...] = x
    u = jnp.concatenate(
        [ir_ref[...] * asrc_ref[...].reshape(1, -1),
         adst_ref[...].reshape(1, -1),
         jnp.zeros((RP - R - 1, D), f32)], axis=0)
    p_ref[...] = jnp.dot(x, u.T, preferred_element_type=f32)

    @pl.when(pl.program_id(0) == 0)
    def _():
        rf_ref[...] = jnp.dot(ir_ref[...], wrel_ref[...],
                              preferred_element_type=f32)


def _dense_stage(ent, init_embed, text_W, text_b, text_factor, fusion_weights,
                 fusion_bias, init_rel, att_src, att_dst, conv_Wrel):
    rep = lambda shape: pl.BlockSpec(shape, lambda i: (0,) * len(shape))
    return pl.pallas_call(
        _dense_body,
        grid=(NB,),
        in_specs=[
            pl.BlockSpec((BN, 768), lambda i: (i, 0)),
            pl.BlockSpec((BN, D), lambda i: (i, 0)),
            rep((768, D)), rep((D,)), rep((16, D + 1, D)), rep((1, 16)),
            rep((1, D)), rep((R, D)), rep((D,)), rep((D,)), rep((D, D)),
        ],
        out_specs=[
            pl.BlockSpec((BN, D), lambda i: (i, 0)),
            pl.BlockSpec((BN, RP), lambda i: (i, 0)),
            pl.BlockSpec((R, D), lambda i: (0, 0)),
        ],
        out_shape=[
            jax.ShapeDtypeStruct((N, D), jnp.float32),
            jax.ShapeDtypeStruct((N, RP), jnp.float32),
            jax.ShapeDtypeStruct((R, D), jnp.float32),
        ],
    )(ent, init_embed, text_W, text_b, text_factor, fusion_weights,
      fusion_bias, init_rel, att_src, att_dst, conv_Wrel)


_MESH = plsc.VectorSubcoreMesh(core_axis_name="c", subcore_axis_name="s")


def _slot_scratch():
    return [
        pltpu.VMEM((3, BK), jnp.int32),
        pltpu.VMEM((BK,), jnp.float32),
        pltpu.VMEM((BK,), jnp.int32),
        pltpu.VMEM((BK,), jnp.int32),
        pltpu.VMEM((BK,), jnp.int32),
        pltpu.VMEM((BK,), jnp.float32),
        pltpu.VMEM((BK,), jnp.float32),
        pltpu.VMEM((BK, D), jnp.float32),
        pltpu.VMEM((BK, D), jnp.float32),
        pltpu.VMEM((BK, DW), jnp.float32),
        pltpu.SemaphoreType.DMA,
        pltpu.SemaphoreType.DMA,
        pltpu.SemaphoreType.DMA,
        pltpu.SemaphoreType.DMA,
        pltpu.SemaphoreType.DMA,
        pltpu.SemaphoreType.DMA,
    ]


@functools.partial(
    pl.kernel, mesh=_MESH,
    compiler_params=pltpu.CompilerParams(use_tc_tiling_on_sc=False),
    out_type=[pltpu.HBM((NC, NPAD, D), jnp.float32),
              pltpu.HBM((NC, NPAD, DW), jnp.float32)],
    scratch_types=[
        pltpu.VMEM_SHARED((R, D), jnp.float32),
        pltpu.VMEM_SHARED((NPAD, D), jnp.float32),
        pltpu.VMEM_SHARED((NPAD, DW), jnp.float32),
    ] + _slot_scratch() + _slot_scratch(),
)
def _edge_pass(ed_hbm, en_hbm, pflat_hbm, x_hbm, rel_hbm, agg_hbm, den_hbm,
               rel_sh, agg_sh, den_sh, *slots):
    f32 = jnp.float32
    cid = lax.axis_index("c")
    sid = lax.axis_index("s")
    wid = sid * NC + cid
    iota = lax.iota(jnp.int32, 16)
    S = [slots[:16], slots[16:]]

    @pl.when(sid == 0)
    def _():
        pltpu.sync_copy(rel_hbm, rel_sh)

    xb0, eb0 = S[0][7], S[0][9]

    def _zxrow(r, _):
        for j in range(D // 16):
            xb0[r, pl.ds(16 * j, 16)] = jnp.zeros((16,), f32)
        eb0[r, pl.ds(0, DW)] = jnp.zeros((16,), f32)
        return _
    lax.fori_loop(0, BK, _zxrow, None)
    base = sid * ROWS
    for i in range(ROWS // BK):
        pltpu.sync_copy(xb0, agg_sh.at[pl.ds(base + i * BK, BK)])
        pltpu.sync_copy(eb0, den_sh.at[pl.ds(base + i * BK, BK)])
    rem = ROWS % BK
    if rem:
        off = base + (ROWS // BK) * BK
        pltpu.sync_copy(xb0.at[pl.ds(0, rem)], agg_sh.at[pl.ds(off, rem)])
        pltpu.sync_copy(eb0.at[pl.ds(0, rem)], den_sh.at[pl.ds(off, rem)])
    plsc.subcore_barrier()

    def _wait_scat(s):
        ed, en, di, i1, i2, g1, g2, xb, rb, eb = S[s][:10]
        sa, sd = S[s][14], S[s][15]
        pltpu.make_async_copy(xb, agg_sh.at[di], sa).wait()
        pltpu.make_async_copy(eb, den_sh.at[di], sd).wait()

    def _prefetch(s, bid):
        ed, en, di, i1, i2, g1, g2, xb, rb, eb = S[s][:10]
        pltpu.sync_copy(ed_hbm.at[:, pl.ds(bid * BK, BK)], ed)
        pltpu.sync_copy(en_hbm.at[pl.ds(bid * BK, BK)], en)
        for g in range(BK // 16):
            sl = pl.ds(16 * g, 16)
            s16 = ed[0, sl]
            d16 = ed[1, sl]
            t16 = ed[2, sl]
            di[sl] = d16
            i1[sl] = s16 * RP + t16
            i2[sl] = d16 * RP + R
        pltpu.async_copy(pflat_hbm.at[i1], g1, S[s][10])
        pltpu.async_copy(pflat_hbm.at[i2], g2, S[s][11])
        pltpu.async_copy(x_hbm.at[ed.at[0]], xb, S[s][12])
        pltpu.async_copy(rel_sh.at[ed.at[2]], rb, S[s][13])

    def _process(s):
        ed, en, di, i1, i2, g1, g2, xb, rb, eb = S[s][:10]
        pltpu.make_async_copy(pflat_hbm.at[i1], g1, S[s][10]).wait()
        pltpu.make_async_copy(pflat_hbm.at[i2], g2, S[s][11]).wait()
        pltpu.make_async_copy(x_hbm.at[ed.at[0]], xb, S[s][12]).wait()
        pltpu.make_async_copy(rel_sh.at[ed.at[2]], rb, S[s][13]).wait()

        @plsc.parallel_loop(0, BK // 16, 1, unroll=1)
        def _group(g):
            k0 = pl.multiple_of(16 * g, 16)
            sl = pl.ds(k0, 16)
            sc = g1[sl] + g2[sl]
            sc = jnp.where(sc >= 0.0, sc, 0.2 * sc)
            e16 = jnp.exp(sc)
            c16 = e16 * en[sl]
            for i in range(16):
                k = k0 + i
                ck = jnp.full((16,), c16[i], f32)
                for j in range(D // 16):
                    slj = pl.ds(16 * j, 16)
                    xb[k, slj] = (ck * xb[k, slj]) * rb[k, slj]
                eb[k, pl.ds(0, DW)] = jnp.where(iota == 0, e16[i], 0.0)
        pltpu.async_copy(xb, agg_sh.at[di], S[s][14], add=True)
        pltpu.async_copy(eb, den_sh.at[di], S[s][15], add=True)

    nextra = (wid < EXTRA).astype(jnp.int32)
    _prefetch(0, wid)

    def _pair(p, _):
        @pl.when(p > 0)
        def _():
            _wait_scat(1)
        _prefetch(1, (2 * p + 1) * NW + wid)
        _process(0)

        @pl.when((2 * p + 2 < FULL) | ((2 * p + 2 == FULL) & (nextra == 1)))
        def _():
            _wait_scat(0)
            _prefetch(0, (2 * p + 2) * NW + wid)
        _process(1)
        return _
    lax.fori_loop(0, PAIRS, _pair, None)

    @pl.when(nextra == 1)
    def _():
        _process(0)
    _wait_scat(0)
    _wait_scat(1)

    plsc.subcore_barrier()
    pltpu.sync_copy(agg_sh.at[pl.ds(base, ROWS)],
                    agg_hbm.at[cid, pl.ds(base, ROWS)])
    pltpu.sync_copy(den_sh.at[pl.ds(base, ROWS)],
                    den_hbm.at[cid, pl.ds(base, ROWS)])


def _out_body(a_ref, d_ref, w_ref, b_ref, o_ref):
    f32 = jnp.float32
    num = a_ref[0] + a_ref[1]
    den = d_ref[0, :, :1] + d_ref[1, :, :1] + 1e-16
    o_ref[...] = jnp.tanh(
        jnp.dot(num / den, w_ref[...], preferred_element_type=f32)
        + b_ref[...])


def _out_stage(agg, den, conv_W, conv_b):
    return pl.pallas_call(
        _out_body,
        grid=(NB,),
        in_specs=[
            pl.BlockSpec((NC, BN, D), lambda i: (0, i, 0)),
            pl.BlockSpec((NC, BN, DW), lambda i: (0, i, 0)),
            pl.BlockSpec((D, D), lambda i: (0, 0)),
            pl.BlockSpec((D,), lambda i: (0,)),
        ],
        out_specs=pl.BlockSpec((BN, D), lambda i: (i, 0)),
        out_shape=jax.ShapeDtypeStruct((N, D), jnp.float32),
    )(agg, den, conv_W, conv_b)


_B = 1024
_BW = _B // NW


@functools.partial(
    pl.kernel, mesh=_MESH,
    compiler_params=pltpu.CompilerParams(use_tc_tiling_on_sc=False),
    out_type=[jax.ShapeDtypeStruct((_B, D), jnp.float32),
              jax.ShapeDtypeStruct((_B, D), jnp.float32)],
    scratch_types=[
        pltpu.VMEM((_BW,), jnp.int32),
        pltpu.VMEM((_BW, D), jnp.float32),
        pltpu.VMEM((_BW,), jnp.int32),
        pltpu.VMEM((_BW, D), jnp.float32),
        pltpu.SemaphoreType.DMA,
        pltpu.SemaphoreType.DMA,
    ],
)
def _gather_pass(xout_hbm, rfull_hbm, subj_hbm, rel_hbm, o1_hbm, o2_hbm,
                 i1_v, r1_v, i2_v, r2_v, sem0, sem1):
    cid = lax.axis_index("c")
    sid = lax.axis_index("s")
    wid = sid * NC + cid
    b0 = wid * _BW
    pltpu.sync_copy(subj_hbm.at[pl.ds(b0, _BW)], i1_v)
    pltpu.sync_copy(rel_hbm.at[pl.ds(b0, _BW)], i2_v)
    d1 = pltpu.async_copy(xout_hbm.at[i1_v], r1_v, sem0)
    d2 = pltpu.async_copy(rfull_hbm.at[i2_v], r2_v, sem1)
    d1.wait()
    d2.wait()
    pltpu.sync_copy(r1_v, o1_hbm.at[pl.ds(b0, _BW)])
    pltpu.sync_copy(r2_v, o2_hbm.at[pl.ds(b0, _BW)])


def kernel(edge_index, edge_type, subj, rel, edge_norm, init_embed,
           ent2textvector, text_W, text_b, text_factor, fusion_weights,
           fusion_bias, init_rel, conv_W, conv_b, conv_Wrel, att_src, att_dst):
    x, p, r_full = _dense_stage(
        ent2textvector, init_embed, text_W, text_b, text_factor,
        fusion_weights, fusion_bias, init_rel, att_src, att_dst, conv_Wrel)
    ed = jnp.concatenate([edge_index, edge_type[None]], axis=0)
    agg, den = _edge_pass(ed, edge_norm, p.reshape(-1), x, init_rel)
    x_out = _out_stage(agg, den, conv_W, conv_b)
    o1, o2 = _gather_pass(x_out, r_full, subj, rel)
    return (o1, o2, x_out)


# speedup vs baseline: 1.5081x; 1.4456x over previous
import functools

import jax
import jax.numpy as jnp
from jax import lax
from jax.experimental import pallas as pl
from jax.experimental.pallas import tpu as pltpu
from jax.experimental.pallas import tpu_sc as plsc

N = 10000
E = 320000
D = 128
R = 400
RP = 512
W = 144
NB = 10
BN = N // NB
NC, NS = 2, 16
NW = NC * NS
BK = 64
NBLK = E // BK
FULL = NBLK // NW
EXTRA = NBLK % NW
PAIRS = FULL // 2
NPAD = 10112
ROWS = NPAD // NS
DW = 16


def _dense_body(ent_ref, ie_ref, tw_ref, tb_ref, tf_ref, fw_ref, fb_ref,
                ir_ref, asrc_ref, adst_ref, wrel_ref,
                x_ref, p_ref, rf_ref):
    f32 = jnp.float32
    tf = tf_ref[...]
    fw = fw_ref[...]
    wf = jnp.sum(fw.reshape(-1, 1, 1) * tf, axis=0)
    w2 = jnp.dot(tw_ref[...], wf[1:], preferred_element_type=f32)
    c0 = wf[0] + jnp.dot(tb_ref[...].reshape(1, -1), wf[1:],
                         preferred_element_type=f32)[0] + fb_ref[...][0]
    fused = jnp.dot(ent_ref[...], w2, preferred_element_type=f32) + c0
    x = ie_ref[...] * fused
    x_ref[...] = x
    u = jnp.concatenate(
        [ir_ref[...] * asrc_ref[...].reshape(1, -1),
         adst_ref[...].reshape(1, -1),
         jnp.zeros((RP - R - 1, D), f32)], axis=0)
    p_ref[...] = jnp.dot(x, u.T, preferred_element_type=f32)

    @pl.when(pl.program_id(0) == 0)
    def _():
        rf_ref[...] = jnp.dot(ir_ref[...], wrel_ref[...],
                              preferred_element_type=f32)


def _dense_stage(ent, init_embed, text_W, text_b, text_factor, fusion_weights,
                 fusion_bias, init_rel, att_src, att_dst, conv_Wrel):
    rep = lambda shape: pl.BlockSpec(shape, lambda i: (0,) * len(shape))
    return pl.pallas_call(
        _dense_body,
        grid=(NB,),
        in_specs=[
            pl.BlockSpec((BN, 768), lambda i: (i, 0)),
            pl.BlockSpec((BN, D), lambda i: (i, 0)),
            rep((768, D)), rep((D,)), rep((16, D + 1, D)), rep((1, 16)),
            rep((1, D)), rep((R, D)), rep((D,)), rep((D,)), rep((D, D)),
        ],
        out_specs=[
            pl.BlockSpec((BN, D), lambda i: (i, 0)),
            pl.BlockSpec((BN, RP), lambda i: (i, 0)),
            pl.BlockSpec((R, D), lambda i: (0, 0)),
        ],
        out_shape=[
            jax.ShapeDtypeStruct((N, D), jnp.float32),
            jax.ShapeDtypeStruct((N, RP), jnp.float32),
            jax.ShapeDtypeStruct((R, D), jnp.float32),
        ],
    )(ent, init_embed, text_W, text_b, text_factor, fusion_weights,
      fusion_bias, init_rel, att_src, att_dst, conv_Wrel)


_MESH = plsc.VectorSubcoreMesh(core_axis_name="c", subcore_axis_name="s")


def _slot_scratch():
    return [
        pltpu.VMEM((3, BK), jnp.int32),
        pltpu.VMEM((BK,), jnp.float32),
        pltpu.VMEM((BK,), jnp.int32),
        pltpu.VMEM((2 * BK,), jnp.int32),
        pltpu.VMEM((2 * BK,), jnp.float32),
        pltpu.VMEM((BK, D), jnp.float32),
        pltpu.VMEM((BK, D), jnp.float32),
        pltpu.VMEM((BK, DW), jnp.float32),
        pltpu.SemaphoreType.DMA,
        pltpu.SemaphoreType.DMA,
        pltpu.SemaphoreType.DMA,
        pltpu.SemaphoreType.DMA,
        pltpu.SemaphoreType.DMA,
        pltpu.SemaphoreType.DMA,
        pltpu.SemaphoreType.DMA,
    ]


@functools.partial(
    pl.kernel, mesh=_MESH,
    compiler_params=pltpu.CompilerParams(use_tc_tiling_on_sc=False),
    out_type=[pltpu.HBM((NC, NPAD, D), jnp.float32),
              pltpu.HBM((NC, NPAD, DW), jnp.float32)],
    scratch_types=[
        pltpu.VMEM_SHARED((R, D), jnp.float32),
        pltpu.VMEM_SHARED((NPAD, D), jnp.float32),
        pltpu.VMEM_SHARED((NPAD, DW), jnp.float32),
    ] + _slot_scratch() + _slot_scratch(),
)
def _edge_pass(ed_hbm, en_hbm, pflat_hbm, x_hbm, rel_hbm, agg_hbm, den_hbm,
               rel_sh, agg_sh, den_sh, *slots):
    f32 = jnp.float32
    cid = lax.axis_index("c")
    sid = lax.axis_index("s")
    wid = sid * NC + cid
    iota = lax.iota(jnp.int32, 16)
    S = [slots[:15], slots[15:]]

    @pl.when(sid == 0)
    def _():
        pltpu.sync_copy(rel_hbm, rel_sh)

    xb0, eb0 = S[0][5], S[0][7]

    def _zxrow(r, _):
        for j in range(D // 16):
            xb0[r, pl.ds(16 * j, 16)] = jnp.zeros((16,), f32)
        eb0[r, pl.ds(0, DW)] = jnp.zeros((16,), f32)
        return _
    lax.fori_loop(0, BK, _zxrow, None)
    base = sid * ROWS
    for i in range(ROWS // BK):
        pltpu.sync_copy(xb0, agg_sh.at[pl.ds(base + i * BK, BK)])
        pltpu.sync_copy(eb0, den_sh.at[pl.ds(base + i * BK, BK)])
    rem = ROWS % BK
    if rem:
        off = base + (ROWS // BK) * BK
        pltpu.sync_copy(xb0.at[pl.ds(0, rem)], agg_sh.at[pl.ds(off, rem)])
        pltpu.sync_copy(eb0.at[pl.ds(0, rem)], den_sh.at[pl.ds(off, rem)])
    plsc.subcore_barrier()

    def _wait_scat(s):
        ed, en, di, i12, g12, xb, rb, eb = S[s][:8]
        pltpu.make_async_copy(xb, agg_sh.at[di], S[s][11]).wait()
        pltpu.make_async_copy(eb, den_sh.at[di], S[s][12]).wait()

    def _stage_issue(s, bid):
        ed, en = S[s][0], S[s][1]
        pltpu.async_copy(ed_hbm.at[:, pl.ds(bid * BK, BK)], ed, S[s][13])
        pltpu.async_copy(en_hbm.at[pl.ds(bid * BK, BK)], en, S[s][14])

    def _prefetch(s, bid):
        ed, en, di, i12, g12, xb, rb, eb = S[s][:8]
        pltpu.make_async_copy(ed_hbm.at[:, pl.ds(bid * BK, BK)], ed,
                              S[s][13]).wait()
        pltpu.make_async_copy(en_hbm.at[pl.ds(bid * BK, BK)], en,
                              S[s][14]).wait()
        for g in range(BK // 16):
            sl = pl.ds(16 * g, 16)
            s16 = ed[0, sl]
            d16 = ed[1, sl]
            t16 = ed[2, sl]
            di[sl] = d16
            i12[sl] = s16 * RP + t16
            i12[pl.ds(BK + 16 * g, 16)] = d16 * RP + R
        pltpu.async_copy(pflat_hbm.at[i12], g12, S[s][8])
        pltpu.async_copy(x_hbm.at[ed.at[0]], xb, S[s][9])
        pltpu.async_copy(rel_sh.at[ed.at[2]], rb, S[s][10])

    def _process(s, nxt):
        ed, en, di, i12, g12, xb, rb, eb = S[s][:8]
        pltpu.make_async_copy(pflat_hbm.at[i12], g12, S[s][8]).wait()
        pltpu.make_async_copy(x_hbm.at[ed.at[0]], xb, S[s][9]).wait()
        pltpu.make_async_copy(rel_sh.at[ed.at[2]], rb, S[s][10]).wait()

        def _group(g, _):
            k0 = pl.multiple_of(16 * g, 16)
            sl = pl.ds(k0, 16)
            sc = g12[sl] + g12[pl.ds(BK + 16 * g, 16)]
            sc = jnp.where(sc >= 0.0, sc, 0.2 * sc)
            e16 = jnp.exp(sc)
            c16 = e16 * en[sl]
            for i in range(16):
                k = k0 + i
                ck = jnp.full((16,), c16[i], f32)
                for j in range(D // 16):
                    slj = pl.ds(16 * j, 16)
                    xb[k, slj] = (ck * xb[k, slj]) * rb[k, slj]
                eb[k, pl.ds(0, DW)] = jnp.where(iota == 0, e16[i], 0.0)
            return _
        lax.fori_loop(0, BK // 16, _group, None)
        if nxt is not None:
            bid_next, cond = nxt

            @pl.when(cond)
            def _():
                _stage_issue(s, bid_next)
        pltpu.async_copy(xb, agg_sh.at[di], S[s][11], add=True)
        pltpu.async_copy(eb, den_sh.at[di], S[s][12], add=True)

    nextra = (wid < EXTRA).astype(jnp.int32)
    _stage_issue(0, wid)
    _stage_issue(1, NW + wid)
    _prefetch(0, wid)

    def _pair(p, _):
        @pl.when(p > 0)
        def _():
            _wait_scat(1)
        _prefetch(1, (2 * p + 1) * NW + wid)
        g0 = (2 * p + 2 < FULL) | ((2 * p + 2 == FULL) & (nextra == 1))
        _process(0, ((2 * p + 2) * NW + wid, g0))

        @pl.when(g0)
        def _():
            _wait_scat(0)
            _prefetch(0, (2 * p + 2) * NW + wid)
        _process(1, ((2 * p + 3) * NW + wid, 2 * p + 3 < FULL))
        return _
    lax.fori_loop(0, PAIRS, _pair, None)

    @pl.when(nextra == 1)
    def _():
        _process(0, None)
    _wait_scat(0)
    _wait_scat(1)

    plsc.subcore_barrier()
    pltpu.sync_copy(agg_sh.at[pl.ds(base, ROWS)],
                    agg_hbm.at[cid, pl.ds(base, ROWS)])
    pltpu.sync_copy(den_sh.at[pl.ds(base, ROWS)],
                    den_hbm.at[cid, pl.ds(base, ROWS)])


def _out_body(a_ref, d_ref, w_ref, b_ref, o_ref):
    f32 = jnp.float32
    num = a_ref[0] + a_ref[1]
    den = d_ref[0, :, :1] + d_ref[1, :, :1] + 1e-16
    o_ref[...] = jnp.tanh(
        jnp.dot(num / den, w_ref[...], preferred_element_type=f32)
        + b_ref[...])


def _out_stage(agg, den, conv_W, conv_b):
    return pl.pallas_call(
        _out_body,
        grid=(NB,),
        in_specs=[
            pl.BlockSpec((NC, BN, D), lambda i: (0, i, 0)),
            pl.BlockSpec((NC, BN, DW), lambda i: (0, i, 0)),
            pl.BlockSpec((D, D), lambda i: (0, 0)),
            pl.BlockSpec((D,), lambda i: (0,)),
        ],
        out_specs=pl.BlockSpec((BN, D), lambda i: (i, 0)),
        out_shape=jax.ShapeDtypeStruct((N, D), jnp.float32),
    )(agg, den, conv_W, conv_b)


_B = 1024
_BW = _B // NW


@functools.partial(
    pl.kernel, mesh=_MESH,
    compiler_params=pltpu.CompilerParams(use_tc_tiling_on_sc=False),
    out_type=[jax.ShapeDtypeStruct((_B, D), jnp.float32),
              jax.ShapeDtypeStruct((_B, D), jnp.float32)],
    scratch_types=[
        pltpu.VMEM((_BW,), jnp.int32),
        pltpu.VMEM((_BW, D), jnp.float32),
        pltpu.VMEM((_BW,), jnp.int32),
        pltpu.VMEM((_BW, D), jnp.float32),
        pltpu.SemaphoreType.DMA,
        pltpu.SemaphoreType.DMA,
    ],
)
def _gather_pass(xout_hbm, rfull_hbm, subj_hbm, rel_hbm, o1_hbm, o2_hbm,
                 i1_v, r1_v, i2_v, r2_v, sem0, sem1):
    cid = lax.axis_index("c")
    sid = lax.axis_index("s")
    wid = sid * NC + cid
    b0 = wid * _BW
    pltpu.sync_copy(subj_hbm.at[pl.ds(b0, _BW)], i1_v)
    pltpu.sync_copy(rel_hbm.at[pl.ds(b0, _BW)], i2_v)
    d1 = pltpu.async_copy(xout_hbm.at[i1_v], r1_v, sem0)
    d2 = pltpu.async_copy(rfull_hbm.at[i2_v], r2_v, sem1)
    d1.wait()
    d2.wait()
    pltpu.sync_copy(r1_v, o1_hbm.at[pl.ds(b0, _BW)])
    pltpu.sync_copy(r2_v, o2_hbm.at[pl.ds(b0, _BW)])


def kernel(edge_index, edge_type, subj, rel, edge_norm, init_embed,
           ent2textvector, text_W, text_b, text_factor, fusion_weights,
           fusion_bias, init_rel, conv_W, conv_b, conv_Wrel, att_src, att_dst):
    x, p, r_full = _dense_stage(
        ent2textvector, init_embed, text_W, text_b, text_factor,
        fusion_weights, fusion_bias, init_rel, att_src, att_dst, conv_Wrel)
    ed = jnp.concatenate([edge_index, edge_type[None]], axis=0)
    agg, den = _edge_pass(ed, edge_norm, p.reshape(-1), x, init_rel)
    x_out = _out_stage(agg, den, conv_W, conv_b)
    o1, o2 = _gather_pass(x_out, r_full, subj, rel)
    return (o1, o2, x_out)
